# Initial kernel scaffold; baseline (speedup 1.0000x reference)
import jax
import jax.numpy as jnp
from jax.experimental import pallas as pl


def kernel(x, y, w_fc, w_fc1, w_fc2):
    raise NotImplementedError("write your pallas kernel here")


# speedup vs baseline: 1.0353x; 1.0353x over previous
import numpy as np
import jax
import jax.numpy as jnp
from jax.experimental import pallas as pl
from jax.experimental.pallas import tpu as pltpu

_HI = jax.lax.Precision.HIGHEST


def _pick_g(h, w):
    for g in range(1, h + 1):
        if h % g == 0 and 128 <= g * w <= 512 and (g * w) % 128 == 0:
            return g
    for g in range(1, h + 1):
        if h % g == 0 and g * w >= 128:
            return g
    return h


def _pool_mat(g, w):
    pr = np.arange(g)[:, None, None, None]
    j = np.arange(w)[None, :, None, None]
    di = np.arange(2)[None, None, :, None]
    dj = np.arange(2)[None, None, None, :]
    u = ((2 * pr + di) * (2 * w) + 2 * j + dj)
    v = np.broadcast_to(pr * w + j, u.shape)
    m = np.zeros((4 * g * w, g * w), np.float32)
    m[u.ravel(), v.ravel()] = 0.25
    return m


def _make_body(hw):
    inv_hw = np.float32(1.0 / hw)

    def body(x_ref, y_ref, p_ref, s_ref, e_ref, wfx_ref, wfy_ref,
             w1_ref, w2_ref, o_ref):
        x = x_ref[0]
        y = y_ref[0]
        yp = jnp.dot(y, p_ref[...], preferred_element_type=jnp.float32)

        xs = jnp.sum(x, axis=1, keepdims=True)
        ys = jnp.sum(yp, axis=1, keepdims=True)
        rows = jnp.concatenate([xs, ys], axis=1)
        gap = jnp.dot(s_ref[...], rows, precision=_HI,
                      preferred_element_type=jnp.float32) * inv_hw

        common = jnp.maximum(
            jnp.dot(wfx_ref[...], gap[:, 0:1], precision=_HI,
                    preferred_element_type=jnp.float32) +
            jnp.dot(wfy_ref[...], gap[:, 1:2], precision=_HI,
                    preferred_element_type=jnp.float32),
            0.0)
        xw = jax.nn.sigmoid(
            jnp.dot(w1_ref[...], common, precision=_HI,
                    preferred_element_type=jnp.float32))
        yw = jax.nn.sigmoid(
            jnp.dot(w2_ref[...], common, precision=_HI,
                    preferred_element_type=jnp.float32))

        grow = jnp.dot(e_ref[...], jnp.concatenate([xw, yw], axis=1),
                       precision=_HI,
                       preferred_element_type=jnp.float32)
        o_ref[0] = x * grow[:, 0:1] + grow[:, 1:2] * yp

    return body


@jax.jit
def kernel(x, y, w_fc, w_fc1, w_fc2):
    B, C, H, W = x.shape
    assert y.shape == (B, C, 2 * H, 2 * W)
    hidden = w_fc.shape[1]

    G = _pick_g(H, W)
    reps = H // G
    RB = C * reps
    GW = G * W

    xg = x.reshape(B, RB, GW).astype(jnp.float32)
    yg = y.reshape(B, RB, 4 * GW).astype(jnp.float32)
    pmat = jnp.asarray(_pool_mat(G, W))
    seg = np.zeros((C, RB), np.float32)
    seg[np.repeat(np.arange(C), reps), np.arange(RB)] = 1.0
    smat = jnp.asarray(seg)
    emat = jnp.asarray(seg.T.copy())
    wf = w_fc.astype(jnp.float32)
    wfxT = wf[:C].T
    wfyT = wf[C:].T
    w1T = w_fc1.astype(jnp.float32).T
    w2T = w_fc2.astype(jnp.float32).T

    out = pl.pallas_call(
        _make_body(H * W),
        grid=(B,),
        in_specs=[
            pl.BlockSpec((1, RB, GW), lambda b: (b, 0, 0)),
            pl.BlockSpec((1, RB, 4 * GW), lambda b: (b, 0, 0)),
            pl.BlockSpec((4 * GW, GW), lambda b: (0, 0)),
            pl.BlockSpec((C, RB), lambda b: (0, 0)),
            pl.BlockSpec((RB, C), lambda b: (0, 0)),
            pl.BlockSpec((hidden, C), lambda b: (0, 0)),
            pl.BlockSpec((hidden, C), lambda b: (0, 0)),
            pl.BlockSpec((C, hidden), lambda b: (0, 0)),
            pl.BlockSpec((C, hidden), lambda b: (0, 0)),
        ],
        out_specs=pl.BlockSpec((1, RB, GW), lambda b: (b, 0, 0)),
        out_shape=jax.ShapeDtypeStruct((B, RB, GW), jnp.float32),
        compiler_params=pltpu.CompilerParams(
            dimension_semantics=("parallel",),
            vmem_limit_bytes=48 * 1024 * 1024),
    )(xg, yg, pmat, smat, emat, wfxT, wfyT, w1T, w2T)

    return out.reshape(B, C, H, W)


# speedup vs baseline: 1.2914x; 1.2475x over previous
import numpy as np
import jax
import jax.numpy as jnp
from jax.experimental import pallas as pl
from jax.experimental.pallas import tpu as pltpu

_HI = jax.lax.Precision.HIGHEST


def _pick_g(h, w):
    for g in range(1, h + 1):
        if h % g == 0 and 128 <= g * w <= 512 and (g * w) % 128 == 0:
            return g
    for g in range(1, h + 1):
        if h % g == 0 and g * w >= 128:
            return g
    return h


def _pool_mat(g, w):
    pr = np.arange(g)[:, None, None, None]
    j = np.arange(w)[None, :, None, None]
    di = np.arange(2)[None, None, :, None]
    dj = np.arange(2)[None, None, None, :]
    u = ((2 * pr + di) * (2 * w) + 2 * j + dj)
    v = np.broadcast_to(pr * w + j, u.shape)
    m = np.zeros((4 * g * w, g * w), np.float32)
    m[u.ravel(), v.ravel()] = 0.25
    return m


def _make_body(hw):
    inv_hw = np.float32(1.0 / hw)

    def body(x_ref, y_ref, p_ref, s_ref, e_ref, wfx_ref, wfy_ref,
             w1_ref, w2_ref, o_ref):
        x = x_ref[0]
        y = y_ref[0]
        yp = jnp.dot(y, p_ref[...], preferred_element_type=jnp.float32)

        xs = jnp.sum(x, axis=1, keepdims=True)
        ys = jnp.sum(yp, axis=1, keepdims=True)
        rows = jnp.concatenate([xs, ys], axis=1)
        gap = jnp.dot(s_ref[...], rows,
                      preferred_element_type=jnp.float32) * inv_hw

        common = jnp.maximum(
            jnp.dot(wfx_ref[...], gap[:, 0:1], precision=_HI,
                    preferred_element_type=jnp.float32) +
            jnp.dot(wfy_ref[...], gap[:, 1:2], precision=_HI,
                    preferred_element_type=jnp.float32),
            0.0)
        xw = jax.nn.sigmoid(
            jnp.dot(w1_ref[...], common, precision=_HI,
                    preferred_element_type=jnp.float32))
        yw = jax.nn.sigmoid(
            jnp.dot(w2_ref[...], common, precision=_HI,
                    preferred_element_type=jnp.float32))

        grow = jnp.dot(e_ref[...], jnp.concatenate([xw, yw], axis=1),
                       preferred_element_type=jnp.float32)
        o_ref[0] = x * grow[:, 0:1] + grow[:, 1:2] * yp

    return body


@jax.jit
def kernel(x, y, w_fc, w_fc1, w_fc2):
    B, C, H, W = x.shape
    assert y.shape == (B, C, 2 * H, 2 * W)
    hidden = w_fc.shape[1]

    G = _pick_g(H, W)
    reps = H // G
    RB = C * reps
    GW = G * W

    xg = x.reshape(B, RB, GW).astype(jnp.float32)
    yg = y.reshape(B, RB, 4 * GW).astype(jnp.float32)
    pmat = jnp.asarray(_pool_mat(G, W))
    seg = np.zeros((C, RB), np.float32)
    seg[np.repeat(np.arange(C), reps), np.arange(RB)] = 1.0
    smat = jnp.asarray(seg)
    emat = jnp.asarray(seg.T.copy())
    wf = w_fc.astype(jnp.float32)
    wfxT = wf[:C].T
    wfyT = wf[C:].T
    w1T = w_fc1.astype(jnp.float32).T
    w2T = w_fc2.astype(jnp.float32).T

    out = pl.pallas_call(
        _make_body(H * W),
        grid=(B,),
        in_specs=[
            pl.BlockSpec((1, RB, GW), lambda b: (b, 0, 0)),
            pl.BlockSpec((1, RB, 4 * GW), lambda b: (b, 0, 0)),
            pl.BlockSpec((4 * GW, GW), lambda b: (0, 0)),
            pl.BlockSpec((C, RB), lambda b: (0, 0)),
            pl.BlockSpec((RB, C), lambda b: (0, 0)),
            pl.BlockSpec((hidden, C), lambda b: (0, 0)),
            pl.BlockSpec((hidden, C), lambda b: (0, 0)),
            pl.BlockSpec((C, hidden), lambda b: (0, 0)),
            pl.BlockSpec((C, hidden), lambda b: (0, 0)),
        ],
        out_specs=pl.BlockSpec((1, RB, GW), lambda b: (b, 0, 0)),
        out_shape=jax.ShapeDtypeStruct((B, RB, GW), jnp.float32),
        compiler_params=pltpu.CompilerParams(
            dimension_semantics=("parallel",),
            vmem_limit_bytes=48 * 1024 * 1024),
    )(xg, yg, pmat, smat, emat, wfxT, wfyT, w1T, w2T)

    return out.reshape(B, C, H, W)


# speedup vs baseline: 1.4695x; 1.1379x over previous
import numpy as np
import jax
import jax.numpy as jnp
from jax.experimental import pallas as pl
from jax.experimental.pallas import tpu as pltpu

_HI = jax.lax.Precision.HIGHEST


def _make_body(c, hh, ww):
    inv_hw = np.float32(1.0 / (hh * ww))

    def body(x_ref, y_ref, wfx_ref, wfy_ref, w1_ref, w2_ref, o_ref):
        x = x_ref[0]

        v = y_ref[0, :, 0::2, :] + y_ref[0, :, 1::2, :]
        v = v.reshape(c * hh, 2 * ww)
        ev = 2 * jax.lax.broadcasted_iota(jnp.int32, (c * hh, ww), 1)
        yp = (jnp.take_along_axis(v, ev, axis=1) +
              jnp.take_along_axis(v, ev + 1, axis=1)) \
            * np.float32(0.25)
        yp = yp.reshape(c, hh, ww)

        xs = jnp.sum(x, axis=(1, 2), keepdims=True)[:, :, 0]
        ys = jnp.sum(yp, axis=(1, 2), keepdims=True)[:, :, 0]

        common = jnp.maximum(
            jnp.dot(wfx_ref[...], xs * inv_hw, precision=_HI,
                    preferred_element_type=jnp.float32) +
            jnp.dot(wfy_ref[...], ys * inv_hw, precision=_HI,
                    preferred_element_type=jnp.float32),
            0.0)
        xw = jax.nn.sigmoid(
            jnp.dot(w1_ref[...], common, precision=_HI,
                    preferred_element_type=jnp.float32))
        yw = jax.nn.sigmoid(
            jnp.dot(w2_ref[...], common, precision=_HI,
                    preferred_element_type=jnp.float32))

        o_ref[0] = x * xw[:, :, None] + yw[:, :, None] * yp

    return body


@jax.jit
def kernel(x, y, w_fc, w_fc1, w_fc2):
    B, C, H, W = x.shape
    assert y.shape == (B, C, 2 * H, 2 * W)
    hidden = w_fc.shape[1]

    wf = w_fc.astype(jnp.float32)
    wfxT = wf[:C].T
    wfyT = wf[C:].T
    w1T = w_fc1.astype(jnp.float32).T
    w2T = w_fc2.astype(jnp.float32).T

    return pl.pallas_call(
        _make_body(C, H, W),
        grid=(B,),
        in_specs=[
            pl.BlockSpec((1, C, H, W), lambda b: (b, 0, 0, 0)),
            pl.BlockSpec((1, C, 2 * H, 2 * W), lambda b: (b, 0, 0, 0)),
            pl.BlockSpec((hidden, C), lambda b: (0, 0)),
            pl.BlockSpec((hidden, C), lambda b: (0, 0)),
            pl.BlockSpec((C, hidden), lambda b: (0, 0)),
            pl.BlockSpec((C, hidden), lambda b: (0, 0)),
        ],
        out_specs=pl.BlockSpec((1, C, H, W), lambda b: (b, 0, 0, 0)),
        out_shape=jax.ShapeDtypeStruct((B, C, H, W), jnp.float32),
        compiler_params=pltpu.CompilerParams(
            dimension_semantics=("parallel",),
            vmem_limit_bytes=48 * 1024 * 1024),
    )(x.astype(jnp.float32), y.astype(jnp.float32), wfxT, wfyT, w1T, w2T)


# speedup vs baseline: 6.7651x; 4.6037x over previous
import numpy as np
import jax
import jax.numpy as jnp
from jax.experimental import pallas as pl
from jax.experimental.pallas import tpu as pltpu

_HI = jax.lax.Precision.HIGHEST


def _make_body(c, hh, ww):
    inv_hw = np.float32(1.0 / (hh * ww))

    def body(x_ref, y_ref, wf_ref, w1_ref, w2_ref, o_ref):
        x = x_ref[0]

        y6 = y_ref[0].reshape(hh, 2, ww, 2, c // 128, 128)
        yp = (y6[:, 0, :, 0] + y6[:, 0, :, 1] +
              y6[:, 1, :, 0] + y6[:, 1, :, 1]) \
            * np.float32(0.25)
        yp = yp.reshape(hh * ww, c)

        ones = jnp.full((1, hh * ww), inv_hw, jnp.float32)
        xg = jnp.dot(ones, x, precision=_HI,
                     preferred_element_type=jnp.float32)
        yg = jnp.dot(ones, yp, precision=_HI,
                     preferred_element_type=jnp.float32)

        feat = jnp.concatenate([xg, yg], axis=1)
        common = jnp.maximum(
            jnp.dot(feat, wf_ref[...], precision=_HI,
                    preferred_element_type=jnp.float32), 0.0)
        xw = jax.nn.sigmoid(
            jnp.dot(common, w1_ref[...], precision=_HI,
                    preferred_element_type=jnp.float32))
        yw = jax.nn.sigmoid(
            jnp.dot(common, w2_ref[...], precision=_HI,
                    preferred_element_type=jnp.float32))

        o_ref[0] = x * xw + yw * yp

    return body


@jax.jit
def kernel(x, y, w_fc, w_fc1, w_fc2):
    B, C, H, W = x.shape
    assert y.shape == (B, C, 2 * H, 2 * W)
    hidden = w_fc.shape[1]

    xt = jax.lax.transpose(x.astype(jnp.float32), (0, 2, 3, 1))
    yt = jax.lax.transpose(y.astype(jnp.float32), (0, 2, 3, 1))

    out = pl.pallas_call(
        _make_body(C, H, W),
        grid=(B,),
        in_specs=[
            pl.BlockSpec((1, H * W, C), lambda b: (b, 0, 0)),
            pl.BlockSpec((1, 4 * H * W, C), lambda b: (b, 0, 0)),
            pl.BlockSpec((2 * C, hidden), lambda b: (0, 0)),
            pl.BlockSpec((hidden, C), lambda b: (0, 0)),
            pl.BlockSpec((hidden, C), lambda b: (0, 0)),
        ],
        out_specs=pl.BlockSpec((1, H * W, C), lambda b: (b, 0, 0)),
        out_shape=jax.ShapeDtypeStruct((B, H * W, C), jnp.float32),
        compiler_params=pltpu.CompilerParams(
            dimension_semantics=("parallel",),
            vmem_limit_bytes=48 * 1024 * 1024),
    )(xt.reshape(B, H * W, C), yt.reshape(B, 4 * H * W, C),
      w_fc.astype(jnp.float32), w_fc1.astype(jnp.float32),
      w_fc2.astype(jnp.float32))

    return jax.lax.transpose(out.reshape(B, H, W, C), (0, 3, 1, 2))


# speedup vs baseline: 7.0070x; 1.0358x over previous
import numpy as np
import jax
import jax.numpy as jnp
from jax.experimental import pallas as pl
from jax.experimental.pallas import tpu as pltpu

_HI = jax.lax.Precision.HIGHEST


def _make_body(c, hh, ww):
    inv_hw = np.float32(1.0 / (hh * ww))

    def body(x_ref, ya_ref, yb_ref, wf_ref, w1_ref, w2_ref, o_ref):
        x = x_ref[0]

        def taps(y_ref):
            y6 = y_ref[0].reshape(hh // 2, 2, ww, 2, c // 128, 128)
            return (y6[:, 0, :, 0] + y6[:, 0, :, 1] +
                    y6[:, 1, :, 0] + y6[:, 1, :, 1])

        yp = (jnp.concatenate([taps(ya_ref), taps(yb_ref)], axis=0)
              * np.float32(0.25)).reshape(hh * ww, c)

        ones = jnp.full((1, hh * ww), inv_hw, jnp.float32)
        xg = jnp.dot(ones, x, precision=_HI,
                     preferred_element_type=jnp.float32)
        yg = jnp.dot(ones, yp, precision=_HI,
                     preferred_element_type=jnp.float32)

        feat = jnp.concatenate([xg, yg], axis=1)
        common = jnp.maximum(
            jax.lax.dot_general(feat, wf_ref[...],
                                (((1,), (1,)), ((), ())), precision=_HI,
                                preferred_element_type=jnp.float32),
            0.0)
        xw = jax.nn.sigmoid(
            jnp.dot(common, w1_ref[...], precision=_HI,
                    preferred_element_type=jnp.float32))
        yw = jax.nn.sigmoid(
            jnp.dot(common, w2_ref[...], precision=_HI,
                    preferred_element_type=jnp.float32))

        o_ref[0] = x * xw + yw * yp

    return body


@jax.jit
def kernel(x, y, w_fc, w_fc1, w_fc2):
    B, C, H, W = x.shape
    assert y.shape == (B, C, 2 * H, 2 * W)
    hidden = w_fc.shape[1]

    xt = jax.lax.transpose(x.astype(jnp.float32), (0, 2, 3, 1))
    yt = jax.lax.transpose(y.astype(jnp.float32), (0, 2, 3, 1))
    yr = yt.reshape(B, 4 * H * W, C)

    out = pl.pallas_call(
        _make_body(C, H, W),
        grid=(B,),
        in_specs=[
            pl.BlockSpec((1, H * W, C), lambda b: (b, 0, 0)),
            pl.BlockSpec((1, 2 * H * W, C), lambda b: (b, 0, 0)),
            pl.BlockSpec((1, 2 * H * W, C), lambda b: (b, 1, 0)),
            pl.BlockSpec((hidden, 2 * C), lambda b: (0, 0)),
            pl.BlockSpec((hidden, C), lambda b: (0, 0)),
            pl.BlockSpec((hidden, C), lambda b: (0, 0)),
        ],
        out_specs=pl.BlockSpec((1, H * W, C), lambda b: (b, 0, 0)),
        out_shape=jax.ShapeDtypeStruct((B, H * W, C), jnp.float32),
        compiler_params=pltpu.CompilerParams(
            dimension_semantics=("parallel",),
            vmem_limit_bytes=48 * 1024 * 1024),
    )(xt.reshape(B, H * W, C),
      yr, yr,
      jax.lax.transpose(w_fc.astype(jnp.float32), (1, 0)),
      w_fc1.astype(jnp.float32), w_fc2.astype(jnp.float32))

    return jax.lax.transpose(out.reshape(B, H, W, C), (0, 3, 1, 2))
